# baseline (speedup 1.0000x reference)
import functools

import jax
import jax.numpy as jnp
from jax import lax
from jax.experimental import pallas as pl
from jax.experimental.pallas import tpu as pltpu
from jax.experimental.pallas import tpu_sc as plsc

_K = 64


def _encode_body(x_ref, w_ref, be_ref, bd_ref, o_ref):
    xc = x_ref[...] - bd_ref[...]
    acc = jnp.dot(xc, w_ref[...], preferred_element_type=jnp.float32)
    o_ref[...] = jnp.maximum(acc + be_ref[...], 0.0)


def _topk_body(k, pre_ref, z_ref):
    pre = pre_ref[...]
    bits = jax.lax.bitcast_convert_type(pre, jnp.int32)
    br, nc, nl = pre.shape

    def count(sel16):
        part = jnp.sum(sel16, axis=2, dtype=jnp.bfloat16)
        return jnp.sum(part.astype(jnp.float32), axis=1).astype(
            jnp.int32).reshape(br, 1, 1)

    one16 = jnp.bfloat16(1)
    zero16 = jnp.bfloat16(0)

    hi16 = (bits >> 16).astype(jnp.int16)
    lo16 = ((bits & 0xFFFF) - 32768).astype(jnp.int16)

    hi0 = jnp.max(bits, axis=(1, 2)).reshape(br, 1, 1) >> 16
    lo0 = jnp.zeros_like(hi0)

    def hstep(_, carry):
        lo, hi = carry
        mid = lo + ((hi - lo + 1) >> 1)
        cnt = count(jnp.where(hi16 >= mid.astype(jnp.int16), one16, zero16))
        ok = cnt >= k
        return jnp.where(ok, mid, lo), jnp.where(ok, hi, mid - 1)

    hstar, _ = jax.lax.fori_loop(0, 15, hstep, (lo0, hi0))
    h16 = hstar.astype(jnp.int16)

    c_hi1 = count(jnp.where(hi16 > h16, one16, zero16))
    loM = jnp.where(hi16 == h16, lo16, jnp.int16(-32768))

    def lstep(_, carry):
        lo, hi = carry
        mid = lo + ((hi - lo + 1) >> 1)
        cnt = c_hi1 + count(
            jnp.where(loM >= (mid - 32768).astype(jnp.int16), one16, zero16))
        ok = cnt >= k
        return jnp.where(ok, mid, lo), jnp.where(ok, hi, mid - 1)

    lzero = jnp.zeros_like(hstar)
    lstar, _ = jax.lax.fori_loop(
        0, 16, lstep, (lzero, jnp.full_like(hstar, 65535)))
    ls16 = (lstar - 32768).astype(jnp.int16)
    t = (hstar << 16) + lstar

    c_gt = c_hi1 + count(jnp.where(loM > ls16, one16, zero16))
    m = k - c_gt
    tie16 = loM == ls16
    col = (jax.lax.broadcasted_iota(jnp.int32, pre.shape, 1) * nl
           + jax.lax.broadcasted_iota(jnp.int32, pre.shape, 2))
    col16 = (col - 32768).astype(jnp.int16)
    tcol = jnp.where(tie16, col16, jnp.int16(32767))

    def istep(_, carry):
        lo2, hi2 = carry
        mid = (lo2 + hi2) >> 1
        cq = count(jnp.where(tcol < (mid - 32768).astype(jnp.int16),
                             one16, zero16))
        ok = cq >= m
        return jnp.where(ok, lo2, mid), jnp.where(ok, mid, hi2)

    lo2_0 = jnp.zeros_like(t)
    hi2_0 = jnp.full_like(t, nc * nl)
    _, cut = jax.lax.fori_loop(0, 15, istep, (lo2_0, hi2_0))

    sel = (bits > t) | (tie16 & (col < cut))
    z_ref[...] = jnp.where(sel, pre, 0.0)


def _sc_decode_body(rows_per_worker, z_hbm, wd_hbm, bd_hbm, out_hbm,
                    zrow, idxb, valb, wb0, wb1, acc, bdv, sem0, sem1, semz):
    nc = 2
    wid = lax.axis_index("s") * nc + lax.axis_index("c")
    d_in = acc.shape[0]
    n = zrow.shape[0]
    nvec = n // 16
    lanes = lax.iota(jnp.int32, 16)
    zero16 = jnp.zeros((16,), jnp.float32)

    pltpu.sync_copy(bd_hbm, bdv)

    for r in range(rows_per_worker):
        row = wid * rows_per_worker + r
        pltpu.async_copy(z_hbm.at[row], zrow, semz).wait()

        for q in range(_K // 16):
            idxb[pl.ds(q * 16, 16)] = jnp.zeros((16,), jnp.int32)
            valb[pl.ds(q * 16, 16)] = zero16

        def scan_group(g, p):
            vs = [zrow[pl.ds(g * 128 + v * 16, 16)] for v in range(8)]
            tot = vs[0]
            for v in range(1, 8):
                tot = tot + vs[v]

            def hit(p):
                for v in range(8):
                    x = vs[v]
                    msk = x > 0.0
                    cs = plsc.cumsum(jnp.where(msk, 1, 0))
                    pos = p + cs - 1
                    cols = g * 128 + v * 16 + lanes
                    plsc.store_scatter(idxb, [pos], cols, mask=msk)
                    plsc.store_scatter(valb, [pos], x, mask=msk)
                    p = p + cs[15]
                return p

            any_cnt = plsc.cumsum(jnp.where(tot > 0.0, 1, 0))[15]
            return lax.cond(any_cnt > 0, hit, lambda p: p, p)

        lax.fori_loop(0, nvec // 8, scan_group, jnp.int32(0))

        bufs = (wb0, wb1)
        sems = (sem0, sem1)
        cps = [None, None, None, None]
        cps[0] = pltpu.async_copy(wd_hbm.at[idxb.at[pl.ds(0, 16)]], bufs[0], sems[0])
        for c in range(4):
            if c < 3:
                cps[c + 1] = pltpu.async_copy(
                    wd_hbm.at[idxb.at[pl.ds((c + 1) * 16, 16)]],
                    bufs[(c + 1) % 2], sems[(c + 1) % 2])
            cps[c].wait()
            wb = bufs[c % 2]
            vv = valb[pl.ds(c * 16, 16)]
            vals = [vv[g] for g in range(16)]

            def acc_step(j, _, c=c, wb=wb, vals=vals):
                s = pl.ds(j * 16, 16)
                t = vals[0] * wb[0, s]
                for g in range(1, 16):
                    t = t + vals[g] * wb[g, s]
                if c == 0:
                    acc[s] = t
                elif c == 3:
                    acc[s] = acc[s] + t + bdv[s]
                else:
                    acc[s] = acc[s] + t
                return 0

            lax.fori_loop(0, d_in // 16, acc_step, 0)

        pltpu.sync_copy(acc, out_hbm.at[row])


def kernel(x, W_enc, b_enc, W_dec, b_dec):
    B, D_IN = x.shape
    D_SAE = W_enc.shape[1]
    be2 = b_enc.reshape(1, D_SAE)
    bd2 = b_dec.reshape(1, D_IN)

    BN = 1024
    pre = pl.pallas_call(
        _encode_body,
        grid=(D_SAE // BN,),
        in_specs=[
            pl.BlockSpec((B, D_IN), lambda j: (0, 0)),
            pl.BlockSpec((D_IN, BN), lambda j: (0, j)),
            pl.BlockSpec((1, BN), lambda j: (0, j)),
            pl.BlockSpec((1, D_IN), lambda j: (0, 0)),
        ],
        out_specs=pl.BlockSpec((B, BN), lambda j: (0, j)),
        out_shape=jax.ShapeDtypeStruct((B, D_SAE), jnp.float32),
        compiler_params=pltpu.CompilerParams(
            dimension_semantics=("parallel",)),
    )(x, W_enc, be2, bd2)

    BR = 16
    NCH = D_SAE // 128
    z = pl.pallas_call(
        functools.partial(_topk_body, _K),
        grid=(B // BR,),
        in_specs=[pl.BlockSpec((BR, NCH, 128), lambda i: (i, 0, 0))],
        out_specs=pl.BlockSpec((BR, NCH, 128), lambda i: (i, 0, 0)),
        out_shape=jax.ShapeDtypeStruct((B, NCH, 128), jnp.float32),
        compiler_params=pltpu.CompilerParams(
            dimension_semantics=("parallel",)),
    )(pre.reshape(B, NCH, 128))
    z = z.reshape(B, D_SAE)

    rpw = B // 32
    x_hat = pl.kernel(
        functools.partial(_sc_decode_body, rpw),
        out_type=jax.ShapeDtypeStruct((B, D_IN), jnp.float32),
        mesh=plsc.VectorSubcoreMesh(core_axis_name="c", subcore_axis_name="s"),
        compiler_params=pltpu.CompilerParams(needs_layout_passes=False),
        scratch_types=[
            pltpu.VMEM((D_SAE,), jnp.float32),
            pltpu.VMEM((_K,), jnp.int32),
            pltpu.VMEM((_K,), jnp.float32),
            pltpu.VMEM((16, D_IN), jnp.float32),
            pltpu.VMEM((16, D_IN), jnp.float32),
            pltpu.VMEM((D_IN,), jnp.float32),
            pltpu.VMEM((D_IN,), jnp.float32),
            pltpu.SemaphoreType.DMA,
            pltpu.SemaphoreType.DMA,
            pltpu.SemaphoreType.DMA,
        ],
    )(z, W_dec, b_dec)

    return (x_hat, z)


# speedup vs baseline: 1.7618x; 1.7618x over previous
import functools

import jax
import jax.numpy as jnp
from jax import lax
from jax.experimental import pallas as pl
from jax.experimental.pallas import tpu as pltpu
from jax.experimental.pallas import tpu_sc as plsc

_K = 64


def _encode_body(x_ref, w_ref, be_ref, bd_ref, o_ref):
    xc = x_ref[...] - bd_ref[...]
    acc = jnp.dot(xc, w_ref[...], preferred_element_type=jnp.float32)
    o_ref[...] = jnp.maximum(acc + be_ref[...], 0.0)


def _topk_body(k, pre_ref, z_ref):
    pre = pre_ref[...]
    bits = jax.lax.bitcast_convert_type(pre, jnp.int32)
    n = pre.shape[1]

    hi0 = jnp.max(bits, axis=1, keepdims=True)
    lo0 = jnp.zeros_like(hi0)

    def vstep(_, carry):
        lo, hi = carry
        mid = lo + ((hi - lo + 1) >> 1)
        cnt = jnp.sum((bits >= mid).astype(jnp.int32), axis=1, keepdims=True)
        ok = cnt >= k
        return jnp.where(ok, mid, lo), jnp.where(ok, hi, mid - 1)

    t, _ = jax.lax.fori_loop(0, 31, vstep, (lo0, hi0))

    c_gt = jnp.sum((bits > t).astype(jnp.int32), axis=1, keepdims=True)
    m = k - c_gt
    col = jax.lax.broadcasted_iota(jnp.int32, pre.shape, 1)
    tcol = jnp.where(bits == t, col, n)

    def istep(_, carry):
        lo2, hi2 = carry
        mid = (lo2 + hi2) >> 1
        cq = jnp.sum((tcol < mid).astype(jnp.int32), axis=1, keepdims=True)
        ok = cq >= m
        return jnp.where(ok, lo2, mid), jnp.where(ok, mid, hi2)

    lo2_0 = jnp.zeros_like(t)
    hi2_0 = jnp.full_like(t, n)
    _, cut = jax.lax.fori_loop(0, 15, istep, (lo2_0, hi2_0))

    sel = (bits > t) | (tcol < cut)
    z_ref[...] = jnp.where(sel, pre, 0.0)


def _sc_decode_body(rows_per_worker, z_hbm, wd_hbm, bd_hbm, out_hbm,
                    zrow, idxb, valb, wb0, wb1, acc, bdv, sem0, sem1, semz):
    nc = 2
    wid = lax.axis_index("s") * nc + lax.axis_index("c")
    d_in = acc.shape[0]
    n = zrow.shape[0]
    nvec = n // 16
    lanes = lax.iota(jnp.int32, 16)
    zero16 = jnp.zeros((16,), jnp.float32)

    pltpu.sync_copy(bd_hbm, bdv)

    for r in range(rows_per_worker):
        row = wid * rows_per_worker + r
        pltpu.async_copy(z_hbm.at[row], zrow, semz).wait()

        for q in range(_K // 16):
            idxb[pl.ds(q * 16, 16)] = jnp.zeros((16,), jnp.int32)
            valb[pl.ds(q * 16, 16)] = zero16

        def scan_group(g, p):
            vs = [zrow[pl.ds(g * 128 + v * 16, 16)] for v in range(8)]
            t01, t23 = vs[0] + vs[1], vs[2] + vs[3]
            t45, t67 = vs[4] + vs[5], vs[6] + vs[7]
            tot = (t01 + t23) + (t45 + t67)

            def hit(p):
                for v in range(8):
                    x = vs[v]
                    msk = x > 0.0
                    cs = plsc.cumsum(jnp.where(msk, 1, 0))
                    pos = p + cs - 1
                    cols = g * 128 + v * 16 + lanes
                    plsc.store_scatter(idxb, [pos], cols, mask=msk)
                    plsc.store_scatter(valb, [pos], x, mask=msk)
                    p = p + cs[15]
                return p

            any_cnt = plsc.cumsum(jnp.where(tot > 0.0, 1, 0))[15]
            return lax.cond(any_cnt > 0, hit, lambda p: p, p)

        lax.fori_loop(0, nvec // 8, scan_group, jnp.int32(0))

        bufs = (wb0, wb1)
        sems = (sem0, sem1)
        cps = [None, None, None, None]
        cps[0] = pltpu.async_copy(wd_hbm.at[idxb.at[pl.ds(0, 16)]], bufs[0], sems[0])
        for c in range(4):
            if c < 3:
                cps[c + 1] = pltpu.async_copy(
                    wd_hbm.at[idxb.at[pl.ds((c + 1) * 16, 16)]],
                    bufs[(c + 1) % 2], sems[(c + 1) % 2])
            cps[c].wait()
            wb = bufs[c % 2]
            vv = valb[pl.ds(c * 16, 16)]
            vals = [vv[g] for g in range(16)]

            def acc_step(j, _, c=c, wb=wb, vals=vals):
                s = pl.ds(j * 16, 16)
                ps = [vals[g] * wb[g, s] for g in range(16)]
                while len(ps) > 1:
                    ps = [ps[i] + ps[i + 1] for i in range(0, len(ps), 2)]
                t = ps[0]
                if c == 0:
                    acc[s] = t
                elif c == 3:
                    acc[s] = acc[s] + t + bdv[s]
                else:
                    acc[s] = acc[s] + t
                return 0

            lax.fori_loop(0, d_in // 16, acc_step, 0, unroll=2)

        pltpu.sync_copy(acc, out_hbm.at[row])


def kernel(x, W_enc, b_enc, W_dec, b_dec):
    B, D_IN = x.shape
    D_SAE = W_enc.shape[1]
    be2 = b_enc.reshape(1, D_SAE)
    bd2 = b_dec.reshape(1, D_IN)

    BN = 1024
    pre = pl.pallas_call(
        _encode_body,
        grid=(D_SAE // BN,),
        in_specs=[
            pl.BlockSpec((B, D_IN), lambda j: (0, 0)),
            pl.BlockSpec((D_IN, BN), lambda j: (0, j)),
            pl.BlockSpec((1, BN), lambda j: (0, j)),
            pl.BlockSpec((1, D_IN), lambda j: (0, 0)),
        ],
        out_specs=pl.BlockSpec((B, BN), lambda j: (0, j)),
        out_shape=jax.ShapeDtypeStruct((B, D_SAE), jnp.float32),
        compiler_params=pltpu.CompilerParams(
            dimension_semantics=("parallel",)),
    )(x, W_enc, be2, bd2)

    BR = 16
    GR = 32
    ngroups = B // GR

    def topk_group(pre_g):
        return pl.pallas_call(
            functools.partial(_topk_body, _K),
            grid=(GR // BR,),
            in_specs=[pl.BlockSpec((BR, D_SAE), lambda i: (i, 0))],
            out_specs=pl.BlockSpec((BR, D_SAE), lambda i: (i, 0)),
            out_shape=jax.ShapeDtypeStruct((GR, D_SAE), jnp.float32),
            compiler_params=pltpu.CompilerParams(
                dimension_semantics=("parallel",)),
        )(pre_g)

    def decode_group(z_g):
        return pl.kernel(
            functools.partial(_sc_decode_body, GR // 32),
            out_type=jax.ShapeDtypeStruct((GR, D_IN), jnp.float32),
            mesh=plsc.VectorSubcoreMesh(
                core_axis_name="c", subcore_axis_name="s"),
            compiler_params=pltpu.CompilerParams(needs_layout_passes=False),
            scratch_types=[
                pltpu.VMEM((D_SAE,), jnp.float32),
                pltpu.VMEM((_K,), jnp.int32),
                pltpu.VMEM((_K,), jnp.float32),
                pltpu.VMEM((16, D_IN), jnp.float32),
                pltpu.VMEM((16, D_IN), jnp.float32),
                pltpu.VMEM((D_IN,), jnp.float32),
                pltpu.VMEM((D_IN,), jnp.float32),
                pltpu.SemaphoreType.DMA,
                pltpu.SemaphoreType.DMA,
                pltpu.SemaphoreType.DMA,
            ],
        )(z_g, W_dec, b_dec)

    zs = [topk_group(lax.slice(pre, (g * GR, 0), ((g + 1) * GR, D_SAE)))
          for g in range(ngroups)]
    xhats = [decode_group(zs[g]) for g in range(ngroups)]

    return (jnp.concatenate(xhats, 0), jnp.concatenate(zs, 0))


# speedup vs baseline: 1.8266x; 1.0367x over previous
import functools

import jax
import jax.numpy as jnp
from jax import lax
from jax.experimental import pallas as pl
from jax.experimental.pallas import tpu as pltpu
from jax.experimental.pallas import tpu_sc as plsc

_K = 64


def _encode_body(x_ref, w_ref, be_ref, bd_ref, o_ref):
    xc = x_ref[...] - bd_ref[...]
    acc = jnp.dot(xc, w_ref[...], preferred_element_type=jnp.float32)
    o_ref[...] = jnp.maximum(acc + be_ref[...], 0.0)


def _topk_body(k, pre_ref, z_ref):
    pre = pre_ref[...]
    bits = jax.lax.bitcast_convert_type(pre, jnp.int32)
    n = pre.shape[1]

    hi0 = jnp.max(bits, axis=1, keepdims=True)
    lo0 = jnp.zeros_like(hi0)

    def vstep(_, carry):
        lo, hi = carry
        mid = lo + ((hi - lo + 1) >> 1)
        cnt = jnp.sum((bits >= mid).astype(jnp.int32), axis=1, keepdims=True)
        ok = cnt >= k
        return jnp.where(ok, mid, lo), jnp.where(ok, hi, mid - 1)

    t, _ = jax.lax.fori_loop(0, 31, vstep, (lo0, hi0))

    c_gt = jnp.sum((bits > t).astype(jnp.int32), axis=1, keepdims=True)
    m = k - c_gt
    col = jax.lax.broadcasted_iota(jnp.int32, pre.shape, 1)
    tcol = jnp.where(bits == t, col, n)

    def istep(_, carry):
        lo2, hi2 = carry
        mid = (lo2 + hi2) >> 1
        cq = jnp.sum((tcol < mid).astype(jnp.int32), axis=1, keepdims=True)
        ok = cq >= m
        return jnp.where(ok, lo2, mid), jnp.where(ok, mid, hi2)

    lo2_0 = jnp.zeros_like(t)
    hi2_0 = jnp.full_like(t, n)
    _, cut = jax.lax.fori_loop(0, 15, istep, (lo2_0, hi2_0))

    sel = (bits > t) | (tcol < cut)
    z_ref[...] = jnp.where(sel, pre, 0.0)


def _sc_decode_body(rows_per_worker, z_hbm, wd_hbm, bd_hbm, out_hbm,
                    zrow, idxb, valb, wb0, wb1, acc, bdv, sem0, sem1, semz):
    nc = 2
    wid = lax.axis_index("s") * nc + lax.axis_index("c")
    d_in = acc.shape[0]
    n = zrow.shape[0]
    nvec = n // 16
    lanes = lax.iota(jnp.int32, 16)
    zero16 = jnp.zeros((16,), jnp.float32)

    pltpu.sync_copy(bd_hbm, bdv)

    for r in range(rows_per_worker):
        row = wid * rows_per_worker + r
        pltpu.async_copy(z_hbm.at[row], zrow, semz).wait()

        for q in range(_K // 16):
            idxb[pl.ds(q * 16, 16)] = jnp.zeros((16,), jnp.int32)
            valb[pl.ds(q * 16, 16)] = zero16

        def scan_group(g, p):
            vs = [zrow[pl.ds(g * 128 + v * 16, 16)] for v in range(8)]
            t01, t23 = vs[0] + vs[1], vs[2] + vs[3]
            t45, t67 = vs[4] + vs[5], vs[6] + vs[7]
            tot = (t01 + t23) + (t45 + t67)

            def hit(p):
                for v in range(8):
                    x = vs[v]
                    msk = x > 0.0
                    cs = plsc.cumsum(jnp.where(msk, 1, 0))
                    pos = p + cs - 1
                    cols = g * 128 + v * 16 + lanes
                    plsc.store_scatter(idxb, [pos], cols, mask=msk)
                    plsc.store_scatter(valb, [pos], x, mask=msk)
                    p = p + cs[15]
                return p

            any_cnt = plsc.cumsum(jnp.where(tot > 0.0, 1, 0))[15]
            return lax.cond(any_cnt > 0, hit, lambda p: p, p)

        lax.fori_loop(0, nvec // 8, scan_group, jnp.int32(0))

        bufs = (wb0, wb1)
        sems = (sem0, sem1)
        cps = [None, None, None, None]
        cps[0] = pltpu.async_copy(wd_hbm.at[idxb.at[pl.ds(0, 16)]], bufs[0], sems[0])
        for c in range(4):
            if c < 3:
                cps[c + 1] = pltpu.async_copy(
                    wd_hbm.at[idxb.at[pl.ds((c + 1) * 16, 16)]],
                    bufs[(c + 1) % 2], sems[(c + 1) % 2])
            cps[c].wait()
            wb = bufs[c % 2]
            vv = valb[pl.ds(c * 16, 16)]
            vals = [vv[g] for g in range(16)]

            def acc_step(j, _, c=c, wb=wb, vals=vals):
                s = pl.ds(j * 16, 16)
                ps = [vals[g] * wb[g, s] for g in range(16)]
                while len(ps) > 1:
                    ps = [ps[i] + ps[i + 1] for i in range(0, len(ps), 2)]
                t = ps[0]
                if c == 0:
                    acc[s] = t
                elif c == 3:
                    acc[s] = acc[s] + t + bdv[s]
                else:
                    acc[s] = acc[s] + t
                return 0

            lax.fori_loop(0, d_in // 16, acc_step, 0, unroll=2)

        pltpu.sync_copy(acc, out_hbm.at[row])


def kernel(x, W_enc, b_enc, W_dec, b_dec):
    B, D_IN = x.shape
    D_SAE = W_enc.shape[1]
    be2 = b_enc.reshape(1, D_SAE)
    bd2 = b_dec.reshape(1, D_IN)

    BN = 1024
    pre = pl.pallas_call(
        _encode_body,
        grid=(D_SAE // BN,),
        in_specs=[
            pl.BlockSpec((B, D_IN), lambda j: (0, 0)),
            pl.BlockSpec((D_IN, BN), lambda j: (0, j)),
            pl.BlockSpec((1, BN), lambda j: (0, j)),
            pl.BlockSpec((1, D_IN), lambda j: (0, 0)),
        ],
        out_specs=pl.BlockSpec((B, BN), lambda j: (0, j)),
        out_shape=jax.ShapeDtypeStruct((B, D_SAE), jnp.float32),
        compiler_params=pltpu.CompilerParams(
            dimension_semantics=("parallel",)),
    )(x, W_enc, be2, bd2)

    BR = 16
    GR = 32
    ngroups = B // GR

    def topk_group(g):
        base = g * (GR // BR)
        return pl.pallas_call(
            functools.partial(_topk_body, _K),
            grid=(GR // BR,),
            in_specs=[pl.BlockSpec((BR, D_SAE), lambda i: (base + i, 0))],
            out_specs=pl.BlockSpec((BR, D_SAE), lambda i: (i, 0)),
            out_shape=jax.ShapeDtypeStruct((GR, D_SAE), jnp.float32),
            compiler_params=pltpu.CompilerParams(
                dimension_semantics=("parallel",)),
        )(pre)

    def decode_group(z_g):
        return pl.kernel(
            functools.partial(_sc_decode_body, GR // 32),
            out_type=jax.ShapeDtypeStruct((GR, D_IN), jnp.float32),
            mesh=plsc.VectorSubcoreMesh(
                core_axis_name="c", subcore_axis_name="s"),
            compiler_params=pltpu.CompilerParams(needs_layout_passes=False),
            scratch_types=[
                pltpu.VMEM((D_SAE,), jnp.float32),
                pltpu.VMEM((_K,), jnp.int32),
                pltpu.VMEM((_K,), jnp.float32),
                pltpu.VMEM((16, D_IN), jnp.float32),
                pltpu.VMEM((16, D_IN), jnp.float32),
                pltpu.VMEM((D_IN,), jnp.float32),
                pltpu.VMEM((D_IN,), jnp.float32),
                pltpu.SemaphoreType.DMA,
                pltpu.SemaphoreType.DMA,
                pltpu.SemaphoreType.DMA,
            ],
        )(z_g, W_dec, b_dec)

    zs = [topk_group(g) for g in range(ngroups)]
    xhats = [decode_group(zs[g]) for g in range(ngroups)]

    return (jnp.concatenate(xhats, 0), jnp.concatenate(zs, 0))


# speedup vs baseline: 1.8358x; 1.0050x over previous
import functools

import jax
import jax.numpy as jnp
from jax import lax
from jax.experimental import pallas as pl
from jax.experimental.pallas import tpu as pltpu
from jax.experimental.pallas import tpu_sc as plsc

_K = 64


def _encode_body(x_ref, w_ref, be_ref, bd_ref, o_ref):
    xc = x_ref[...] - bd_ref[...]
    acc = jnp.dot(xc, w_ref[...], preferred_element_type=jnp.float32)
    o_ref[...] = jnp.maximum(acc + be_ref[...], 0.0)


def _topk_body(k, pre_ref, z_ref):
    pre = pre_ref[...]
    bits = jax.lax.bitcast_convert_type(pre, jnp.int32)
    n = pre.shape[1]

    hi0 = jnp.max(bits, axis=1, keepdims=True)
    lo0 = jnp.zeros_like(hi0)

    def vstep(_, carry):
        lo, hi = carry
        mid = lo + ((hi - lo + 1) >> 1)
        cnt = jnp.sum((bits >= mid).astype(jnp.int32), axis=1, keepdims=True)
        ok = cnt >= k
        return jnp.where(ok, mid, lo), jnp.where(ok, hi, mid - 1)

    t, _ = jax.lax.fori_loop(0, 31, vstep, (lo0, hi0))

    col = jax.lax.broadcasted_iota(jnp.int32, pre.shape, 1)
    tcol = jnp.where(bits > t, -1, jnp.where(bits == t, col, n))

    def istep(_, carry):
        lo2, hi2 = carry
        mid = (lo2 + hi2) >> 1
        cq = jnp.sum((tcol < mid).astype(jnp.int32), axis=1, keepdims=True)
        ok = cq >= k
        return jnp.where(ok, lo2, mid), jnp.where(ok, mid, hi2)

    lo2_0 = jnp.zeros_like(t)
    hi2_0 = jnp.full_like(t, n)
    _, cut = jax.lax.fori_loop(0, 15, istep, (lo2_0, hi2_0))

    z_ref[...] = jnp.where(tcol < cut, pre, 0.0)


def _sc_decode_body(rows_per_worker, z_hbm, wd_hbm, bd_hbm, out_hbm,
                    zrow, idxb, valb, wb0, wb1, acc, bdv, sem0, sem1, semz):
    nc = 2
    wid = lax.axis_index("s") * nc + lax.axis_index("c")
    d_in = acc.shape[0]
    n = zrow.shape[0]
    nvec = n // 16
    lanes = lax.iota(jnp.int32, 16)
    zero16 = jnp.zeros((16,), jnp.float32)

    pltpu.sync_copy(bd_hbm, bdv)

    for r in range(rows_per_worker):
        row = wid * rows_per_worker + r
        pltpu.async_copy(z_hbm.at[row], zrow, semz).wait()

        for q in range(_K // 16):
            idxb[pl.ds(q * 16, 16)] = jnp.zeros((16,), jnp.int32)
            valb[pl.ds(q * 16, 16)] = zero16

        def scan_group(g, p):
            vs = [zrow[pl.ds(g * 128 + v * 16, 16)] for v in range(8)]
            t01, t23 = vs[0] + vs[1], vs[2] + vs[3]
            t45, t67 = vs[4] + vs[5], vs[6] + vs[7]
            tot = (t01 + t23) + (t45 + t67)

            def hit(p):
                for v in range(8):
                    x = vs[v]
                    msk = x > 0.0
                    cs = plsc.cumsum(jnp.where(msk, 1, 0))
                    pos = p + cs - 1
                    cols = g * 128 + v * 16 + lanes
                    plsc.store_scatter(idxb, [pos], cols, mask=msk)
                    plsc.store_scatter(valb, [pos], x, mask=msk)
                    p = p + cs[15]
                return p

            any_cnt = plsc.cumsum(jnp.where(tot > 0.0, 1, 0))[15]
            return lax.cond(any_cnt > 0, hit, lambda p: p, p)

        lax.fori_loop(0, nvec // 8, scan_group, jnp.int32(0))

        bufs = (wb0, wb1)
        sems = (sem0, sem1)
        cps = [None, None, None, None]
        cps[0] = pltpu.async_copy(wd_hbm.at[idxb.at[pl.ds(0, 16)]], bufs[0], sems[0])
        for c in range(4):
            if c < 3:
                cps[c + 1] = pltpu.async_copy(
                    wd_hbm.at[idxb.at[pl.ds((c + 1) * 16, 16)]],
                    bufs[(c + 1) % 2], sems[(c + 1) % 2])
            cps[c].wait()
            wb = bufs[c % 2]
            vv = valb[pl.ds(c * 16, 16)]
            vals = [vv[g] for g in range(16)]

            def acc_step(j, _, c=c, wb=wb, vals=vals):
                s = pl.ds(j * 16, 16)
                ps = [vals[g] * wb[g, s] for g in range(16)]
                while len(ps) > 1:
                    ps = [ps[i] + ps[i + 1] for i in range(0, len(ps), 2)]
                t = ps[0]
                if c == 0:
                    acc[s] = t
                elif c == 3:
                    acc[s] = acc[s] + t + bdv[s]
                else:
                    acc[s] = acc[s] + t
                return 0

            lax.fori_loop(0, d_in // 16, acc_step, 0, unroll=2)

        pltpu.sync_copy(acc, out_hbm.at[row])


def kernel(x, W_enc, b_enc, W_dec, b_dec):
    B, D_IN = x.shape
    D_SAE = W_enc.shape[1]
    be2 = b_enc.reshape(1, D_SAE)
    bd2 = b_dec.reshape(1, D_IN)

    BN = 1024
    pre = pl.pallas_call(
        _encode_body,
        grid=(D_SAE // BN,),
        in_specs=[
            pl.BlockSpec((B, D_IN), lambda j: (0, 0)),
            pl.BlockSpec((D_IN, BN), lambda j: (0, j)),
            pl.BlockSpec((1, BN), lambda j: (0, j)),
            pl.BlockSpec((1, D_IN), lambda j: (0, 0)),
        ],
        out_specs=pl.BlockSpec((B, BN), lambda j: (0, j)),
        out_shape=jax.ShapeDtypeStruct((B, D_SAE), jnp.float32),
        compiler_params=pltpu.CompilerParams(
            dimension_semantics=("parallel",)),
    )(x, W_enc, be2, bd2)

    BR = 16
    GR = 32
    ngroups = B // GR

    def topk_group(g):
        base = g * (GR // BR)
        return pl.pallas_call(
            functools.partial(_topk_body, _K),
            grid=(GR // BR,),
            in_specs=[pl.BlockSpec((BR, D_SAE), lambda i: (base + i, 0))],
            out_specs=pl.BlockSpec((BR, D_SAE), lambda i: (i, 0)),
            out_shape=jax.ShapeDtypeStruct((GR, D_SAE), jnp.float32),
            compiler_params=pltpu.CompilerParams(
                dimension_semantics=("parallel",)),
        )(pre)

    def decode_group(z_g):
        return pl.kernel(
            functools.partial(_sc_decode_body, GR // 32),
            out_type=jax.ShapeDtypeStruct((GR, D_IN), jnp.float32),
            mesh=plsc.VectorSubcoreMesh(
                core_axis_name="c", subcore_axis_name="s"),
            compiler_params=pltpu.CompilerParams(needs_layout_passes=False),
            scratch_types=[
                pltpu.VMEM((D_SAE,), jnp.float32),
                pltpu.VMEM((_K,), jnp.int32),
                pltpu.VMEM((_K,), jnp.float32),
                pltpu.VMEM((16, D_IN), jnp.float32),
                pltpu.VMEM((16, D_IN), jnp.float32),
                pltpu.VMEM((D_IN,), jnp.float32),
                pltpu.VMEM((D_IN,), jnp.float32),
                pltpu.SemaphoreType.DMA,
                pltpu.SemaphoreType.DMA,
                pltpu.SemaphoreType.DMA,
            ],
        )(z_g, W_dec, b_dec)

    zs = [topk_group(g) for g in range(ngroups)]
    xhats = [decode_group(zs[g]) for g in range(ngroups)]

    return (jnp.concatenate(xhats, 0), jnp.concatenate(zs, 0))


# speedup vs baseline: 1.9172x; 1.0443x over previous
import functools

import jax
import jax.numpy as jnp
from jax import lax
from jax.experimental import pallas as pl
from jax.experimental.pallas import tpu as pltpu
from jax.experimental.pallas import tpu_sc as plsc

_K = 64


def _encode_body(x_ref, w_ref, be_ref, bd_ref, o_ref):
    xc = x_ref[...] - bd_ref[...]
    acc = jnp.dot(xc, w_ref[...], preferred_element_type=jnp.float32)
    o_ref[...] = jnp.maximum(acc + be_ref[...], 0.0)


def _topk_body(k, pre_ref, z_ref):
    pre = pre_ref[...]
    bits = jax.lax.bitcast_convert_type(pre, jnp.int32)
    n = pre.shape[1]

    hi0 = jnp.max(bits, axis=1, keepdims=True)
    lo0 = jnp.zeros_like(hi0)

    def vstep(_, carry):
        lo, hi = carry
        mid = lo + ((hi - lo + 1) >> 1)
        cnt = jnp.sum((bits >= mid).astype(jnp.int32), axis=1, keepdims=True)
        ok = cnt >= k
        return jnp.where(ok, mid, lo), jnp.where(ok, hi, mid - 1)

    t, _ = jax.lax.fori_loop(0, 31, vstep, (lo0, hi0))

    col = jax.lax.broadcasted_iota(jnp.int32, pre.shape, 1)
    tcol = jnp.where(bits > t, -1, jnp.where(bits == t, col, n))

    def istep(_, carry):
        lo2, hi2 = carry
        mid = (lo2 + hi2) >> 1
        cq = jnp.sum((tcol < mid).astype(jnp.int32), axis=1, keepdims=True)
        ok = cq >= k
        return jnp.where(ok, lo2, mid), jnp.where(ok, mid, hi2)

    lo2_0 = jnp.zeros_like(t)
    hi2_0 = jnp.full_like(t, n)
    _, cut = jax.lax.fori_loop(0, 15, istep, (lo2_0, hi2_0))

    z_ref[...] = jnp.where(tcol < cut, pre, 0.0)


def _sc_decode_body(rows_per_worker, z_hbm, wd_hbm, bd_hbm, out_hbm,
                    zrow, idxb, valb, wb0, wb1, acc, bdv, sem0, sem1, semz):
    nc = 2
    wid = lax.axis_index("s") * nc + lax.axis_index("c")
    d_in = acc.shape[0]
    n = zrow.shape[0]
    nvec = n // 16
    lanes = lax.iota(jnp.int32, 16)
    zero16 = jnp.zeros((16,), jnp.float32)

    pltpu.sync_copy(bd_hbm, bdv)

    for r in range(rows_per_worker):
        row = wid * rows_per_worker + r
        pltpu.async_copy(z_hbm.at[row], zrow, semz).wait()

        for q in range(_K // 16):
            idxb[pl.ds(q * 16, 16)] = jnp.zeros((16,), jnp.int32)
            valb[pl.ds(q * 16, 16)] = zero16

        def scan_group(g, p):
            vs = [zrow[pl.ds(g * 128 + v * 16, 16)] for v in range(8)]
            t01, t23 = vs[0] + vs[1], vs[2] + vs[3]
            t45, t67 = vs[4] + vs[5], vs[6] + vs[7]
            tot = (t01 + t23) + (t45 + t67)

            def hit(p):
                for v in range(8):
                    x = vs[v]
                    msk = x > 0.0
                    cs = plsc.cumsum(jnp.where(msk, 1, 0))
                    pos = p + cs - 1
                    cols = g * 128 + v * 16 + lanes
                    plsc.store_scatter(idxb, [pos], cols, mask=msk)
                    plsc.store_scatter(valb, [pos], x, mask=msk)
                    p = p + cs[15]
                return p

            any_cnt = plsc.cumsum(jnp.where(tot > 0.0, 1, 0))[15]
            return lax.cond(any_cnt > 0, hit, lambda p: p, p)

        lax.fori_loop(0, nvec // 8, scan_group, jnp.int32(0))

        bufs = (wb0, wb1)
        sems = (sem0, sem1)
        cps = [None, None, None, None]
        cps[0] = pltpu.async_copy(wd_hbm.at[idxb.at[pl.ds(0, 16)]], bufs[0], sems[0])
        for c in range(4):
            if c < 3:
                cps[c + 1] = pltpu.async_copy(
                    wd_hbm.at[idxb.at[pl.ds((c + 1) * 16, 16)]],
                    bufs[(c + 1) % 2], sems[(c + 1) % 2])
            cps[c].wait()
            wb = bufs[c % 2]
            vv = valb[pl.ds(c * 16, 16)]
            vals = [vv[g] for g in range(16)]

            def acc_step(j, _, c=c, wb=wb, vals=vals):
                s = pl.ds(j * 16, 16)
                ps = [vals[g] * wb[g, s] for g in range(16)]
                while len(ps) > 1:
                    ps = [ps[i] + ps[i + 1] for i in range(0, len(ps), 2)]
                t = ps[0]
                if c == 0:
                    acc[s] = t
                elif c == 3:
                    acc[s] = acc[s] + t + bdv[s]
                else:
                    acc[s] = acc[s] + t
                return 0

            lax.fori_loop(0, d_in // 16, acc_step, 0, unroll=2)

        pltpu.sync_copy(acc, out_hbm.at[row])


def kernel(x, W_enc, b_enc, W_dec, b_dec):
    B, D_IN = x.shape
    D_SAE = W_enc.shape[1]
    be2 = b_enc.reshape(1, D_SAE)
    bd2 = b_dec.reshape(1, D_IN)

    BN = 1024
    pre = pl.pallas_call(
        _encode_body,
        grid=(D_SAE // BN,),
        in_specs=[
            pl.BlockSpec((B, D_IN), lambda j: (0, 0)),
            pl.BlockSpec((D_IN, BN), lambda j: (0, j)),
            pl.BlockSpec((1, BN), lambda j: (0, j)),
            pl.BlockSpec((1, D_IN), lambda j: (0, 0)),
        ],
        out_specs=pl.BlockSpec((B, BN), lambda j: (0, j)),
        out_shape=jax.ShapeDtypeStruct((B, D_SAE), jnp.float32),
        compiler_params=pltpu.CompilerParams(
            dimension_semantics=("parallel",)),
    )(x, W_enc, be2, bd2)

    BR = 32
    GR = 32
    ngroups = B // GR

    def topk_group(g):
        base = g * (GR // BR)
        return pl.pallas_call(
            functools.partial(_topk_body, _K),
            grid=(GR // BR,),
            in_specs=[pl.BlockSpec((BR, D_SAE), lambda i: (base + i, 0))],
            out_specs=pl.BlockSpec((BR, D_SAE), lambda i: (i, 0)),
            out_shape=jax.ShapeDtypeStruct((GR, D_SAE), jnp.float32),
            compiler_params=pltpu.CompilerParams(
                dimension_semantics=("parallel",)),
        )(pre)

    def decode_group(z_g):
        return pl.kernel(
            functools.partial(_sc_decode_body, GR // 32),
            out_type=jax.ShapeDtypeStruct((GR, D_IN), jnp.float32),
            mesh=plsc.VectorSubcoreMesh(
                core_axis_name="c", subcore_axis_name="s"),
            compiler_params=pltpu.CompilerParams(needs_layout_passes=False),
            scratch_types=[
                pltpu.VMEM((D_SAE,), jnp.float32),
                pltpu.VMEM((_K,), jnp.int32),
                pltpu.VMEM((_K,), jnp.float32),
                pltpu.VMEM((16, D_IN), jnp.float32),
                pltpu.VMEM((16, D_IN), jnp.float32),
                pltpu.VMEM((D_IN,), jnp.float32),
                pltpu.VMEM((D_IN,), jnp.float32),
                pltpu.SemaphoreType.DMA,
                pltpu.SemaphoreType.DMA,
                pltpu.SemaphoreType.DMA,
            ],
        )(z_g, W_dec, b_dec)

    zs = [topk_group(g) for g in range(ngroups)]
    xhats = [decode_group(zs[g]) for g in range(ngroups)]

    return (jnp.concatenate(xhats, 0), jnp.concatenate(zs, 0))
